# baseline (speedup 1.0000x reference)
import functools

import jax
import jax.numpy as jnp
from jax import lax
from jax.experimental import pallas as pl
from jax.experimental.pallas import tpu as pltpu
from jax.experimental.pallas import tpu_sc as plsc

T = 2048
D = 1024
I = 2048
E = 8
K = 2
NP = T * K
BLK = 256
NBLK = NP // BLK + E - 1
NPAD = NBLK * BLK
IC = 512
NIC = I // IC
CH = 256
TB = 256


def _router_kernel(x_ref, wg_ref, ti_ref, tw_ref, cnt_ref, pos_ref, aux_ref,
                   ohs_ref):
    x = x_ref[...]
    wg = wg_ref[...]
    logits = jax.lax.dot_general(x, wg, (((1,), (1,)), ((), ())),
                                 preferred_element_type=jnp.float32)
    m = jnp.max(logits, axis=1, keepdims=True)
    ex = jnp.exp(logits - m)
    probs = ex / jnp.sum(ex, axis=1, keepdims=True)

    eids = jax.lax.broadcasted_iota(jnp.int32, (T, E), 1)
    m1 = jnp.max(probs, axis=1, keepdims=True)
    i1 = jnp.min(jnp.where(probs >= m1, eids, E), axis=1, keepdims=True)
    oh1 = (eids == i1)
    probs2 = jnp.where(oh1, -jnp.inf, probs)
    m2 = jnp.max(probs2, axis=1, keepdims=True)
    i2 = jnp.min(jnp.where(probs2 >= m2, eids, E), axis=1, keepdims=True)
    oh2 = (eids == i2)
    denom = m1 + m2 + 1e-9
    w1 = m1 / denom
    w2 = m2 / denom

    oh1f = oh1.astype(jnp.float32)
    oh2f = oh2.astype(jnp.float32)
    ohs = oh1f + oh2f
    cnt = jnp.sum(ohs, axis=0, keepdims=True)
    p_mean = jnp.mean(probs, axis=0, keepdims=True)
    aux_ref[...] = E * jnp.sum((cnt / T) * p_mean, axis=1, keepdims=True)
    cnt_i = cnt.astype(jnp.int32)
    cnt_ref[...] = cnt_i

    pcnt = ((cnt + (BLK - 1)) // BLK).astype(jnp.float32) * BLK
    r8 = jax.lax.broadcasted_iota(jnp.int32, (E, E), 0)
    c8 = jax.lax.broadcasted_iota(jnp.int32, (E, E), 1)
    lt8 = (r8 < c8).astype(jnp.float32)
    pad_off = jax.lax.dot_general(pcnt, lt8, (((1,), (0,)), ((), ())),
                                  preferred_element_type=jnp.float32)

    ohs_ref[...] = ohs
    rr = jax.lax.broadcasted_iota(jnp.int32, (CH, CH), 0)
    cc = jax.lax.broadcasted_iota(jnp.int32, (CH, CH), 1)
    ltri = (cc < rr).astype(jnp.float32)

    def scan_body(c, carry):
        blk = ohs_ref[pl.ds(c * CH, CH), :]
        pre = jax.lax.dot_general(ltri, blk, (((1,), (0,)), ((), ())),
                                  preferred_element_type=jnp.float32)
        ohs_ref[pl.ds(c * CH, CH), :] = pre + carry
        return carry + jnp.sum(blk, axis=0, keepdims=True)

    jax.lax.fori_loop(0, T // CH, scan_body, jnp.zeros((1, E), jnp.float32))
    cex = ohs_ref[...]

    sel_off1 = jnp.sum(oh1f * pad_off, axis=1, keepdims=True)
    sel_off2 = jnp.sum(oh2f * pad_off, axis=1, keepdims=True)
    rank1 = jnp.sum(oh1f * cex, axis=1, keepdims=True)
    rank2 = jnp.sum(oh2f * cex, axis=1, keepdims=True)
    pos1 = (sel_off1 + rank1).astype(jnp.int32)
    pos2 = (sel_off2 + rank2).astype(jnp.int32)

    ti_ref[...] = jnp.concatenate([i1, i2], axis=1)
    tw_ref[...] = jnp.concatenate([w1, w2], axis=1)
    pos_ref[...] = jnp.concatenate([pos1, pos2], axis=1)


def _router_call(x_flat, w_gate, interpret=False):
    return pl.pallas_call(
        _router_kernel,
        out_shape=[
            jax.ShapeDtypeStruct((T, K), jnp.int32),
            jax.ShapeDtypeStruct((T, K), jnp.float32),
            jax.ShapeDtypeStruct((1, E), jnp.int32),
            jax.ShapeDtypeStruct((T, K), jnp.int32),
            jax.ShapeDtypeStruct((1, 1), jnp.float32),
        ],
        scratch_shapes=[pltpu.VMEM((T, E), jnp.float32)],
        interpret=interpret,
    )(x_flat, w_gate)


NW = 32
RPW = NPAD // NW
GCH = (64, 64, 56)

_SC_MESH = plsc.VectorSubcoreMesh(core_axis_name="c", subcore_axis_name="s")


def _sc_gather_kernel(x_hbm, tok_hbm, out_hbm, idx_v, rows_v, sem):
    wid = lax.axis_index("s") * 2 + lax.axis_index("c")
    base = wid * RPW
    pltpu.sync_copy(tok_hbm.at[pl.ds(base, RPW)], idx_v)
    off = 0
    for size in GCH:
        pltpu.async_copy(x_hbm.at[idx_v.at[pl.ds(off, size)]],
                         rows_v.at[pl.ds(0, size)], sem).wait()
        pltpu.sync_copy(rows_v.at[pl.ds(0, size)],
                        out_hbm.at[pl.ds(base + off, size)])
        off += size


def _sc_gather(x_flat, tok_pad):
    k = functools.partial(
        pl.kernel, mesh=_SC_MESH,
        out_type=jax.ShapeDtypeStruct((NPAD, D), jnp.float32),
        scratch_types=[
            pltpu.VMEM((RPW,), jnp.int32),
            pltpu.VMEM((64, D), jnp.float32),
            pltpu.SemaphoreType.DMA,
        ],
    )(_sc_gather_kernel)
    return k(x_flat, tok_pad)


def _ffn_kernel(be_ref, nb_ref, xs_ref, gu_ref, down_ref, y_ref):
    b = pl.program_id(0)

    @pl.when(b < nb_ref[0])
    def _active():
        xg = xs_ref[...]
        h = jax.lax.dot_general(xg, gu_ref[0], (((1,), (1,)), ((), ())),
                                preferred_element_type=jnp.float32)
        g = h[:, :I]
        u = h[:, I:]
        act = (g / (1.0 + jnp.exp(-g))) * u
        y_ref[...] = jax.lax.dot_general(
            act, down_ref[0], (((1,), (1,)), ((), ())),
            preferred_element_type=jnp.float32)


def _ffn_call(be, nb, xs, egu, ed, interpret=False):
    spec = pltpu.PrefetchScalarGridSpec(
        num_scalar_prefetch=2,
        grid=(NBLK,),
        in_specs=[
            pl.BlockSpec((BLK, D), lambda b, be, nb: (b, 0)),
            pl.BlockSpec((1, 2 * I, D), lambda b, be, nb: (be[b], 0, 0),
                         pipeline_mode=pl.Buffered(buffer_count=1)),
            pl.BlockSpec((1, D, I), lambda b, be, nb: (be[b], 0, 0),
                         pipeline_mode=pl.Buffered(buffer_count=1)),
        ],
        out_specs=pl.BlockSpec((BLK, D), lambda b, be, nb: (b, 0)),
    )
    return pl.pallas_call(
        _ffn_kernel,
        grid_spec=spec,
        out_shape=jax.ShapeDtypeStruct((NPAD, D), jnp.float32),
        interpret=interpret,
    )(be, nb, xs, egu, ed)


def _combine_kernel(pos_ref, twb_ref, y_ref, out_ref):
    tb = pl.program_id(0)

    def body(r, _):
        g = tb * TB + r
        p0 = pos_ref[2 * g]
        p1 = pos_ref[2 * g + 1]
        w0 = jax.lax.bitcast_convert_type(twb_ref[2 * g], jnp.float32)
        w1 = jax.lax.bitcast_convert_type(twb_ref[2 * g + 1], jnp.float32)
        out_ref[pl.ds(r, 1), :] = (y_ref[pl.ds(p0, 1), :] * w0
                                   + y_ref[pl.ds(p1, 1), :] * w1)
        return 0

    jax.lax.fori_loop(0, TB, body, 0)


def _combine_call(pos_flat, tw_flat_bits, y_pad, interpret=False):
    spec = pltpu.PrefetchScalarGridSpec(
        num_scalar_prefetch=2,
        grid=(T // TB,),
        in_specs=[pl.BlockSpec((NPAD, D), lambda tb, pos, tw: (0, 0))],
        out_specs=pl.BlockSpec((TB, D), lambda tb, pos, tw: (tb, 0)),
    )
    return pl.pallas_call(
        _combine_kernel,
        grid_spec=spec,
        out_shape=jax.ShapeDtypeStruct((T, D), jnp.float32),
        interpret=interpret,
    )(pos_flat, tw_flat_bits, y_pad)


def _moe(x, w_gate, egu, ed, interpret=False):
    b_, s_, d_ = x.shape
    x_flat = x.reshape(T, D)
    ti, tw, cnt, pos, aux = _router_call(x_flat, w_gate, interpret=interpret)

    pb = (cnt.reshape(E) + (BLK - 1)) // BLK
    ends = jnp.cumsum(pb)
    nblk_act = ends[E - 1].astype(jnp.int32).reshape(1)
    bids = jnp.arange(NBLK, dtype=jnp.int32)
    eb = jnp.minimum(jnp.sum((bids[:, None] >= ends[None, :]).astype(jnp.int32),
                             axis=1), E - 1).astype(jnp.int32)
    last_e = jnp.max(jnp.where(bids < nblk_act[0], eb, -1))
    be = jnp.where(bids < nblk_act[0], eb, last_e).astype(jnp.int32)
    pos_flat = pos.reshape(NP)
    tok_src = (jnp.arange(NP, dtype=jnp.int32) // K)
    tok_pad = jnp.zeros((NPAD,), jnp.int32).at[pos_flat].set(tok_src)

    xs = _sc_gather(x_flat, tok_pad)
    y_pad = _ffn_call(be, nblk_act, xs, egu, ed, interpret=interpret)

    tw_bits = jax.lax.bitcast_convert_type(tw.reshape(NP), jnp.int32)
    out = _combine_call(pos_flat, tw_bits, y_pad, interpret=interpret)
    return out.reshape(b_, s_, d_), aux[0, 0]


def kernel(x, W_gate, expert_gate_up, expert_down):
    return _moe(x, W_gate, expert_gate_up, expert_down)


# speedup vs baseline: 1.0088x; 1.0088x over previous
import functools

import jax
import jax.numpy as jnp
from jax import lax
from jax.experimental import pallas as pl
from jax.experimental.pallas import tpu as pltpu
from jax.experimental.pallas import tpu_sc as plsc

T = 2048
D = 1024
I = 2048
E = 8
K = 2
NP = T * K
BLK = 256
NBLK = NP // BLK + E - 1
NPAD = NBLK * BLK
IC = 512
NIC = I // IC
CH = 256
TB = 256


def _router_kernel(x_ref, wg_ref, ti_ref, tw_ref, cnt_ref, pos_ref, aux_ref,
                   ohs_ref):
    x = x_ref[...]
    wg = wg_ref[...]
    logits = jax.lax.dot_general(x, wg, (((1,), (1,)), ((), ())),
                                 preferred_element_type=jnp.float32)
    m = jnp.max(logits, axis=1, keepdims=True)
    ex = jnp.exp(logits - m)
    probs = ex / jnp.sum(ex, axis=1, keepdims=True)

    eids = jax.lax.broadcasted_iota(jnp.int32, (T, E), 1)
    m1 = jnp.max(probs, axis=1, keepdims=True)
    i1 = jnp.min(jnp.where(probs >= m1, eids, E), axis=1, keepdims=True)
    oh1 = (eids == i1)
    probs2 = jnp.where(oh1, -jnp.inf, probs)
    m2 = jnp.max(probs2, axis=1, keepdims=True)
    i2 = jnp.min(jnp.where(probs2 >= m2, eids, E), axis=1, keepdims=True)
    oh2 = (eids == i2)
    denom = m1 + m2 + 1e-9
    w1 = m1 / denom
    w2 = m2 / denom

    oh1f = oh1.astype(jnp.float32)
    oh2f = oh2.astype(jnp.float32)
    ohs = oh1f + oh2f
    cnt = jnp.sum(ohs, axis=0, keepdims=True)
    p_mean = jnp.mean(probs, axis=0, keepdims=True)
    aux_ref[...] = E * jnp.sum((cnt / T) * p_mean, axis=1, keepdims=True)
    cnt_i = cnt.astype(jnp.int32)
    cnt_ref[...] = cnt_i

    pcnt = ((cnt + (BLK - 1)) // BLK).astype(jnp.float32) * BLK
    r8 = jax.lax.broadcasted_iota(jnp.int32, (E, E), 0)
    c8 = jax.lax.broadcasted_iota(jnp.int32, (E, E), 1)
    lt8 = (r8 < c8).astype(jnp.float32)
    pad_off = jax.lax.dot_general(pcnt, lt8, (((1,), (0,)), ((), ())),
                                  preferred_element_type=jnp.float32)

    ohs_ref[...] = ohs
    rr = jax.lax.broadcasted_iota(jnp.int32, (CH, CH), 0)
    cc = jax.lax.broadcasted_iota(jnp.int32, (CH, CH), 1)
    ltri = (cc < rr).astype(jnp.float32)

    def scan_body(c, carry):
        blk = ohs_ref[pl.ds(c * CH, CH), :]
        pre = jax.lax.dot_general(ltri, blk, (((1,), (0,)), ((), ())),
                                  preferred_element_type=jnp.float32)
        ohs_ref[pl.ds(c * CH, CH), :] = pre + carry
        return carry + jnp.sum(blk, axis=0, keepdims=True)

    jax.lax.fori_loop(0, T // CH, scan_body, jnp.zeros((1, E), jnp.float32))
    cex = ohs_ref[...]

    sel_off1 = jnp.sum(oh1f * pad_off, axis=1, keepdims=True)
    sel_off2 = jnp.sum(oh2f * pad_off, axis=1, keepdims=True)
    rank1 = jnp.sum(oh1f * cex, axis=1, keepdims=True)
    rank2 = jnp.sum(oh2f * cex, axis=1, keepdims=True)
    pos1 = (sel_off1 + rank1).astype(jnp.int32)
    pos2 = (sel_off2 + rank2).astype(jnp.int32)

    ti_ref[...] = jnp.concatenate([i1, i2], axis=1)
    tw_ref[...] = jnp.concatenate([w1, w2], axis=1)
    pos_ref[...] = jnp.concatenate([pos1, pos2], axis=1)


def _router_call(x_flat, w_gate, interpret=False):
    return pl.pallas_call(
        _router_kernel,
        out_shape=[
            jax.ShapeDtypeStruct((T, K), jnp.int32),
            jax.ShapeDtypeStruct((T, K), jnp.float32),
            jax.ShapeDtypeStruct((1, E), jnp.int32),
            jax.ShapeDtypeStruct((T, K), jnp.int32),
            jax.ShapeDtypeStruct((1, 1), jnp.float32),
        ],
        scratch_shapes=[pltpu.VMEM((T, E), jnp.float32)],
        interpret=interpret,
    )(x_flat, w_gate)


NW = 32
RPW = NPAD // NW
GCH = (64, 64, 56)

_SC_MESH = plsc.VectorSubcoreMesh(core_axis_name="c", subcore_axis_name="s")


def _sc_gather_kernel(x_hbm, tok_hbm, out_hbm, idx_v, rows_v, sem):
    wid = lax.axis_index("s") * 2 + lax.axis_index("c")
    base = wid * RPW
    pltpu.sync_copy(tok_hbm.at[pl.ds(base, RPW)], idx_v)
    off = 0
    for size in GCH:
        pltpu.async_copy(x_hbm.at[idx_v.at[pl.ds(off, size)]],
                         rows_v.at[pl.ds(0, size)], sem).wait()
        pltpu.sync_copy(rows_v.at[pl.ds(0, size)],
                        out_hbm.at[pl.ds(base + off, size)])
        off += size


def _sc_gather(x_flat, tok_pad):
    k = functools.partial(
        pl.kernel, mesh=_SC_MESH,
        out_type=jax.ShapeDtypeStruct((NPAD, D), jnp.float32),
        scratch_types=[
            pltpu.VMEM((RPW,), jnp.int32),
            pltpu.VMEM((64, D), jnp.float32),
            pltpu.SemaphoreType.DMA,
        ],
    )(_sc_gather_kernel)
    return k(x_flat, tok_pad)


def _ffn_kernel(be_ref, nb_ref, xs_ref, gu_ref, down_ref, w_ref, y_ref):
    b = pl.program_id(0)

    @pl.when(b < nb_ref[0])
    def _active():
        xg = xs_ref[...]
        h = jax.lax.dot_general(xg, gu_ref[0], (((1,), (1,)), ((), ())),
                                preferred_element_type=jnp.float32)
        g = h[:, :I]
        u = h[:, I:]
        act = (g / (1.0 + jnp.exp(-g))) * u
        part = jax.lax.dot_general(
            act, down_ref[0], (((1,), (1,)), ((), ())),
            preferred_element_type=jnp.float32)
        y_ref[...] = part * w_ref[...]


def _ffn_call(be, nb, xs, egu, ed, w_pad, interpret=False):
    spec = pltpu.PrefetchScalarGridSpec(
        num_scalar_prefetch=2,
        grid=(NBLK,),
        in_specs=[
            pl.BlockSpec((BLK, D), lambda b, be, nb: (b, 0)),
            pl.BlockSpec((1, 2 * I, D), lambda b, be, nb: (be[b], 0, 0),
                         pipeline_mode=pl.Buffered(buffer_count=1)),
            pl.BlockSpec((1, D, I), lambda b, be, nb: (be[b], 0, 0),
                         pipeline_mode=pl.Buffered(buffer_count=1)),
            pl.BlockSpec((BLK, 1), lambda b, be, nb: (b, 0)),
        ],
        out_specs=pl.BlockSpec((BLK, D), lambda b, be, nb: (b, 0)),
    )
    return pl.pallas_call(
        _ffn_kernel,
        grid_spec=spec,
        out_shape=jax.ShapeDtypeStruct((NPAD, D), jnp.float32),
        interpret=interpret,
    )(be, nb, xs, egu, ed, w_pad)


TPW = T // NW
TCH = 32


def _sc_combine_kernel(y_hbm, p0_hbm, p1_hbm, out_hbm, i0_v, i1_v, a_v, b_v,
                       sem0, sem1):
    wid = lax.axis_index("s") * 2 + lax.axis_index("c")
    base = wid * TPW
    pltpu.sync_copy(p0_hbm.at[pl.ds(base, TPW)], i0_v)
    pltpu.sync_copy(p1_hbm.at[pl.ds(base, TPW)], i1_v)
    for cstart in range(0, TPW, TCH):
        cp0 = pltpu.async_copy(y_hbm.at[i0_v.at[pl.ds(cstart, TCH)]], a_v,
                               sem0)
        cp1 = pltpu.async_copy(y_hbm.at[i1_v.at[pl.ds(cstart, TCH)]], b_v,
                               sem1)
        cp0.wait()
        cp1.wait()

        def row_body(r, _):
            def lane_body(j, _2):
                sl = pl.ds(j * 16, 16)
                a_v[r, sl] = a_v[r, sl] + b_v[r, sl]
                return 0
            return lax.fori_loop(0, D // 16, lane_body, 0)

        lax.fori_loop(0, TCH, row_body, 0)
        pltpu.sync_copy(a_v, out_hbm.at[pl.ds(base + cstart, TCH)])


def _sc_combine(y_pad_scaled, pos0, pos1):
    k = functools.partial(
        pl.kernel, mesh=_SC_MESH,
        out_type=jax.ShapeDtypeStruct((T, D), jnp.float32),
        scratch_types=[
            pltpu.VMEM((TPW,), jnp.int32),
            pltpu.VMEM((TPW,), jnp.int32),
            pltpu.VMEM((TCH, D), jnp.float32),
            pltpu.VMEM((TCH, D), jnp.float32),
            pltpu.SemaphoreType.DMA,
            pltpu.SemaphoreType.DMA,
        ],
    )(_sc_combine_kernel)
    return k(y_pad_scaled, pos0, pos1)


def _moe(x, w_gate, egu, ed, interpret=False):
    b_, s_, d_ = x.shape
    x_flat = x.reshape(T, D)
    ti, tw, cnt, pos, aux = _router_call(x_flat, w_gate, interpret=interpret)

    pb = (cnt.reshape(E) + (BLK - 1)) // BLK
    ends = jnp.cumsum(pb)
    nblk_act = ends[E - 1].astype(jnp.int32).reshape(1)
    bids = jnp.arange(NBLK, dtype=jnp.int32)
    eb = jnp.minimum(jnp.sum((bids[:, None] >= ends[None, :]).astype(jnp.int32),
                             axis=1), E - 1).astype(jnp.int32)
    last_e = jnp.max(jnp.where(bids < nblk_act[0], eb, -1))
    be = jnp.where(bids < nblk_act[0], eb, last_e).astype(jnp.int32)
    pos_flat = pos.reshape(NP)
    tok_src = (jnp.arange(NP, dtype=jnp.int32) // K)
    tok_pad = jnp.zeros((NPAD,), jnp.int32).at[pos_flat].set(tok_src)

    w_pad = jnp.zeros((NPAD,), jnp.float32).at[pos_flat].set(tw.reshape(NP))

    xs = _sc_gather(x_flat, tok_pad)
    y_pad = _ffn_call(be, nblk_act, xs, egu, ed, w_pad.reshape(NPAD, 1),
                      interpret=interpret)
    out = _sc_combine(y_pad, pos[:, 0], pos[:, 1])
    return out.reshape(b_, s_, d_), aux[0, 0]


def kernel(x, W_gate, expert_gate_up, expert_down):
    return _moe(x, W_gate, expert_gate_up, expert_down)


# speedup vs baseline: 1.1512x; 1.1412x over previous
import functools

import jax
import jax.numpy as jnp
from jax import lax
from jax.experimental import pallas as pl
from jax.experimental.pallas import tpu as pltpu
from jax.experimental.pallas import tpu_sc as plsc

T = 2048
D = 1024
I = 2048
E = 8
K = 2
NP = T * K
BLK = 256
NBLK = NP // BLK + E - 1
NPAD = NBLK * BLK
IC = 512
NIC = I // IC
CH = 256
TB = 256


def _router_kernel(x_ref, wg_ref, ti_ref, tw_ref, cnt_ref, pos_ref, aux_ref,
                   ohs_ref):
    x = x_ref[...]
    wg = wg_ref[...]
    logits = jax.lax.dot_general(x, wg, (((1,), (1,)), ((), ())),
                                 preferred_element_type=jnp.float32)
    m = jnp.max(logits, axis=1, keepdims=True)
    ex = jnp.exp(logits - m)
    probs = ex / jnp.sum(ex, axis=1, keepdims=True)

    eids = jax.lax.broadcasted_iota(jnp.int32, (T, E), 1)
    m1 = jnp.max(probs, axis=1, keepdims=True)
    i1 = jnp.min(jnp.where(probs >= m1, eids, E), axis=1, keepdims=True)
    oh1 = (eids == i1)
    probs2 = jnp.where(oh1, -jnp.inf, probs)
    m2 = jnp.max(probs2, axis=1, keepdims=True)
    i2 = jnp.min(jnp.where(probs2 >= m2, eids, E), axis=1, keepdims=True)
    oh2 = (eids == i2)
    denom = m1 + m2 + 1e-9
    w1 = m1 / denom
    w2 = m2 / denom

    oh1f = oh1.astype(jnp.float32)
    oh2f = oh2.astype(jnp.float32)
    ohs = oh1f + oh2f
    cnt = jnp.sum(ohs, axis=0, keepdims=True)
    p_mean = jnp.mean(probs, axis=0, keepdims=True)
    aux_ref[...] = E * jnp.sum((cnt / T) * p_mean, axis=1, keepdims=True)
    cnt_i = cnt.astype(jnp.int32)
    cnt_ref[...] = cnt_i

    pcnt = ((cnt + (BLK - 1)) // BLK).astype(jnp.float32) * BLK
    r8 = jax.lax.broadcasted_iota(jnp.int32, (E, E), 0)
    c8 = jax.lax.broadcasted_iota(jnp.int32, (E, E), 1)
    lt8 = (r8 < c8).astype(jnp.float32)
    pad_off = jax.lax.dot_general(pcnt, lt8, (((1,), (0,)), ((), ())),
                                  preferred_element_type=jnp.float32)

    ohs_ref[...] = ohs
    rr = jax.lax.broadcasted_iota(jnp.int32, (CH, CH), 0)
    cc = jax.lax.broadcasted_iota(jnp.int32, (CH, CH), 1)
    ltri = (cc < rr).astype(jnp.float32)

    def scan_body(c, carry):
        blk = ohs_ref[pl.ds(c * CH, CH), :]
        pre = jax.lax.dot_general(ltri, blk, (((1,), (0,)), ((), ())),
                                  preferred_element_type=jnp.float32)
        ohs_ref[pl.ds(c * CH, CH), :] = pre + carry
        return carry + jnp.sum(blk, axis=0, keepdims=True)

    jax.lax.fori_loop(0, T // CH, scan_body, jnp.zeros((1, E), jnp.float32))
    cex = ohs_ref[...]

    sel_off1 = jnp.sum(oh1f * pad_off, axis=1, keepdims=True)
    sel_off2 = jnp.sum(oh2f * pad_off, axis=1, keepdims=True)
    rank1 = jnp.sum(oh1f * cex, axis=1, keepdims=True)
    rank2 = jnp.sum(oh2f * cex, axis=1, keepdims=True)
    pos1 = (sel_off1 + rank1).astype(jnp.int32)
    pos2 = (sel_off2 + rank2).astype(jnp.int32)

    ti_ref[...] = jnp.concatenate([i1, i2], axis=1)
    tw_ref[...] = jnp.concatenate([w1, w2], axis=1)
    pos_ref[...] = jnp.concatenate([pos1, pos2], axis=1)


def _router_call(x_flat, w_gate, interpret=False):
    return pl.pallas_call(
        _router_kernel,
        out_shape=[
            jax.ShapeDtypeStruct((T, K), jnp.int32),
            jax.ShapeDtypeStruct((T, K), jnp.float32),
            jax.ShapeDtypeStruct((1, E), jnp.int32),
            jax.ShapeDtypeStruct((T, K), jnp.int32),
            jax.ShapeDtypeStruct((1, 1), jnp.float32),
        ],
        scratch_shapes=[pltpu.VMEM((T, E), jnp.float32)],
        interpret=interpret,
    )(x_flat, w_gate)


NW = 32

_SC_MESH = plsc.VectorSubcoreMesh(core_axis_name="c", subcore_axis_name="s")


def _ffn_kernel(be_ref, nb_ref, tok_ref, x_ref, gu_ref, down_ref, w_ref,
                y_ref, xg_ref):
    b = pl.program_id(0)

    @pl.when(b < nb_ref[0])
    def _active():
        def body(r, _):
            t = tok_ref[b * BLK + r]
            xg_ref[pl.ds(r, 1), :] = x_ref[pl.ds(t, 1), :]
            return 0
        jax.lax.fori_loop(0, BLK, body, 0)

        xg = xg_ref[...]
        h = jax.lax.dot_general(xg, gu_ref[0], (((1,), (1,)), ((), ())),
                                preferred_element_type=jnp.float32)
        g = h[:, :I]
        u = h[:, I:]
        act = (g / (1.0 + jnp.exp(-g))) * u
        part = jax.lax.dot_general(
            act, down_ref[0], (((1,), (1,)), ((), ())),
            preferred_element_type=jnp.float32)
        y_ref[...] = part * w_ref[...]


def _ffn_call(be, nb, tok_pad, x_flat, egu, ed, w_pad, interpret=False):
    spec = pltpu.PrefetchScalarGridSpec(
        num_scalar_prefetch=3,
        grid=(NBLK,),
        in_specs=[
            pl.BlockSpec((T, D), lambda b, be, nb, tok: (0, 0)),
            pl.BlockSpec((1, 2 * I, D), lambda b, be, nb, tok: (be[b], 0, 0),
                         pipeline_mode=pl.Buffered(buffer_count=1)),
            pl.BlockSpec((1, D, I), lambda b, be, nb, tok: (be[b], 0, 0),
                         pipeline_mode=pl.Buffered(buffer_count=1)),
            pl.BlockSpec((BLK, 1), lambda b, be, nb, tok: (b, 0)),
        ],
        out_specs=pl.BlockSpec((BLK, D), lambda b, be, nb, tok: (b, 0)),
        scratch_shapes=[pltpu.VMEM((BLK, D), jnp.float32)],
    )
    return pl.pallas_call(
        _ffn_kernel,
        grid_spec=spec,
        out_shape=jax.ShapeDtypeStruct((NPAD, D), jnp.float32),
        interpret=interpret,
    )(be, nb, tok_pad, x_flat, egu, ed, w_pad)


TPW = T // NW
TCH = 32


def _sc_combine_kernel(y_hbm, p0_hbm, p1_hbm, out_hbm, i0_v, i1_v, a_v, b_v,
                       sem0, sem1):
    wid = lax.axis_index("s") * 2 + lax.axis_index("c")
    base = wid * TPW
    pltpu.sync_copy(p0_hbm.at[pl.ds(base, TPW)], i0_v)
    pltpu.sync_copy(p1_hbm.at[pl.ds(base, TPW)], i1_v)
    for cstart in range(0, TPW, TCH):
        cp0 = pltpu.async_copy(y_hbm.at[i0_v.at[pl.ds(cstart, TCH)]], a_v,
                               sem0)
        cp1 = pltpu.async_copy(y_hbm.at[i1_v.at[pl.ds(cstart, TCH)]], b_v,
                               sem1)
        cp0.wait()
        cp1.wait()

        def row_body(r, _):
            def lane_body(j, _2):
                sl = pl.ds(j * 16, 16)
                a_v[r, sl] = a_v[r, sl] + b_v[r, sl]
                return 0
            return lax.fori_loop(0, D // 16, lane_body, 0)

        lax.fori_loop(0, TCH, row_body, 0)
        pltpu.sync_copy(a_v, out_hbm.at[pl.ds(base + cstart, TCH)])


def _sc_combine(y_pad_scaled, pos0, pos1):
    k = functools.partial(
        pl.kernel, mesh=_SC_MESH,
        out_type=jax.ShapeDtypeStruct((T, D), jnp.float32),
        scratch_types=[
            pltpu.VMEM((TPW,), jnp.int32),
            pltpu.VMEM((TPW,), jnp.int32),
            pltpu.VMEM((TCH, D), jnp.float32),
            pltpu.VMEM((TCH, D), jnp.float32),
            pltpu.SemaphoreType.DMA,
            pltpu.SemaphoreType.DMA,
        ],
    )(_sc_combine_kernel)
    return k(y_pad_scaled, pos0, pos1)


def _moe(x, w_gate, egu, ed, interpret=False):
    b_, s_, d_ = x.shape
    x_flat = x.reshape(T, D)
    ti, tw, cnt, pos, aux = _router_call(x_flat, w_gate, interpret=interpret)

    pb = (cnt.reshape(E) + (BLK - 1)) // BLK
    ends = jnp.cumsum(pb)
    nblk_act = ends[E - 1].astype(jnp.int32).reshape(1)
    bids = jnp.arange(NBLK, dtype=jnp.int32)
    eb = jnp.minimum(jnp.sum((bids[:, None] >= ends[None, :]).astype(jnp.int32),
                             axis=1), E - 1).astype(jnp.int32)
    last_e = jnp.max(jnp.where(bids < nblk_act[0], eb, -1))
    be = jnp.where(bids < nblk_act[0], eb, last_e).astype(jnp.int32)
    pos_flat = pos.reshape(NP)
    tok_src = (jnp.arange(NP, dtype=jnp.int32) // K)
    tok_pad = jnp.zeros((NPAD,), jnp.int32).at[pos_flat].set(tok_src)

    w_pad = jnp.zeros((NPAD,), jnp.float32).at[pos_flat].set(tw.reshape(NP))

    y_pad = _ffn_call(be, nblk_act, tok_pad, x_flat, egu, ed,
                      w_pad.reshape(NPAD, 1), interpret=interpret)
    out = _sc_combine(y_pad, pos[:, 0], pos[:, 1])
    return out.reshape(b_, s_, d_), aux[0, 0]


def kernel(x, W_gate, expert_gate_up, expert_down):
    return _moe(x, W_gate, expert_gate_up, expert_down)


# speedup vs baseline: 1.3409x; 1.1648x over previous
import functools

import jax
import jax.numpy as jnp
from jax.experimental import pallas as pl
from jax.experimental.pallas import tpu as pltpu

T = 2048
D = 1024
I = 2048
E = 8
K = 2
NP = T * K
BLK = 256
NBLK = NP // BLK + E - 1
NPAD = NBLK * BLK
IC = 512
NIC = I // IC
CH = 256
TB = 256


def _router_kernel(x_ref, wg_ref, ti_ref, tw_ref, cnt_ref, pos_ref, aux_ref,
                   be_ref, nb_ref, ohs_ref):
    x = x_ref[...]
    wg = wg_ref[...]
    logits = jax.lax.dot_general(x, wg, (((1,), (1,)), ((), ())),
                                 preferred_element_type=jnp.float32)
    m = jnp.max(logits, axis=1, keepdims=True)
    ex = jnp.exp(logits - m)
    probs = ex / jnp.sum(ex, axis=1, keepdims=True)

    eids = jax.lax.broadcasted_iota(jnp.int32, (T, E), 1)
    m1 = jnp.max(probs, axis=1, keepdims=True)
    i1 = jnp.min(jnp.where(probs >= m1, eids, E), axis=1, keepdims=True)
    oh1 = (eids == i1)
    probs2 = jnp.where(oh1, -jnp.inf, probs)
    m2 = jnp.max(probs2, axis=1, keepdims=True)
    i2 = jnp.min(jnp.where(probs2 >= m2, eids, E), axis=1, keepdims=True)
    oh2 = (eids == i2)
    denom = m1 + m2 + 1e-9
    w1 = m1 / denom
    w2 = m2 / denom

    oh1f = oh1.astype(jnp.float32)
    oh2f = oh2.astype(jnp.float32)
    ohs = oh1f + oh2f
    cnt = jnp.sum(ohs, axis=0, keepdims=True)
    p_mean = jnp.mean(probs, axis=0, keepdims=True)
    aux_ref[...] = E * jnp.sum((cnt / T) * p_mean, axis=1, keepdims=True)
    cnt_i = cnt.astype(jnp.int32)
    cnt_ref[...] = cnt_i

    pcnt = ((cnt + (BLK - 1)) // BLK).astype(jnp.float32) * BLK
    r8 = jax.lax.broadcasted_iota(jnp.int32, (E, E), 0)
    c8 = jax.lax.broadcasted_iota(jnp.int32, (E, E), 1)
    lt8 = (r8 < c8).astype(jnp.float32)
    pad_off = jax.lax.dot_general(pcnt, lt8, (((1,), (0,)), ((), ())),
                                  preferred_element_type=jnp.float32)

    ohs_ref[...] = ohs
    rr = jax.lax.broadcasted_iota(jnp.int32, (CH, CH), 0)
    cc = jax.lax.broadcasted_iota(jnp.int32, (CH, CH), 1)
    ltri = (cc < rr).astype(jnp.float32)

    def scan_body(c, carry):
        blk = ohs_ref[pl.ds(c * CH, CH), :]
        pre = jax.lax.dot_general(ltri, blk, (((1,), (0,)), ((), ())),
                                  preferred_element_type=jnp.float32)
        ohs_ref[pl.ds(c * CH, CH), :] = pre + carry
        return carry + jnp.sum(blk, axis=0, keepdims=True)

    jax.lax.fori_loop(0, T // CH, scan_body, jnp.zeros((1, E), jnp.float32))
    cex = ohs_ref[...]

    sel_off1 = jnp.sum(oh1f * pad_off, axis=1, keepdims=True)
    sel_off2 = jnp.sum(oh2f * pad_off, axis=1, keepdims=True)
    rank1 = jnp.sum(oh1f * cex, axis=1, keepdims=True)
    rank2 = jnp.sum(oh2f * cex, axis=1, keepdims=True)
    pos1 = (sel_off1 + rank1).astype(jnp.int32)
    pos2 = (sel_off2 + rank2).astype(jnp.int32)

    ti_ref[...] = jnp.concatenate([i1, i2], axis=1)
    tw_ref[...] = jnp.concatenate([w1, w2], axis=1)
    pos_ref[...] = jnp.concatenate([pos1, pos2], axis=1)

    le8 = (r8 <= c8).astype(jnp.float32)
    ends = jax.lax.dot_general(pcnt / BLK, le8, (((1,), (0,)), ((), ())),
                               preferred_element_type=jnp.float32)
    nbf = ends[:, E - 1:E]
    ends_b = jnp.broadcast_to(ends, (NBLK, E))
    bid_col = jax.lax.broadcasted_iota(jnp.int32, (NBLK, 1), 0).astype(
        jnp.float32)
    eb = jnp.sum((ends_b <= bid_col).astype(jnp.float32), axis=1,
                 keepdims=True)
    active = bid_col < nbf
    last_e = jnp.max(jnp.where(active, eb, -1.0), axis=0, keepdims=True)
    be_ref[...] = jnp.where(active, eb, last_e).astype(jnp.int32)
    nb_ref[...] = nbf.astype(jnp.int32)


def _router_call(x_flat, w_gate, interpret=False):
    return pl.pallas_call(
        _router_kernel,
        out_shape=[
            jax.ShapeDtypeStruct((T, K), jnp.int32),
            jax.ShapeDtypeStruct((T, K), jnp.float32),
            jax.ShapeDtypeStruct((1, E), jnp.int32),
            jax.ShapeDtypeStruct((T, K), jnp.int32),
            jax.ShapeDtypeStruct((1, 1), jnp.float32),
            jax.ShapeDtypeStruct((NBLK, 1), jnp.int32),
            jax.ShapeDtypeStruct((1, 1), jnp.int32),
        ],
        scratch_shapes=[pltpu.VMEM((T, E), jnp.float32)],
        interpret=interpret,
    )(x_flat, w_gate)


def _ffn_kernel(be_ref, nb_ref, tok_ref, x_ref, gu_ref, down_ref,
                y_ref, xg_ref):
    b = pl.program_id(0)

    @pl.when(b < nb_ref[0])
    def _active():
        def body(r, _):
            t = tok_ref[b * BLK + r]
            xg_ref[pl.ds(r, 1), :] = x_ref[pl.ds(t, 1), :]
            return 0
        jax.lax.fori_loop(0, BLK, body, 0)

        xg = xg_ref[...]
        h = jax.lax.dot_general(xg, gu_ref[0], (((1,), (1,)), ((), ())),
                                preferred_element_type=jnp.float32)
        g = h[:, :I]
        u = h[:, I:]
        act = (g / (1.0 + jnp.exp(-g))) * u
        y_ref[...] = jax.lax.dot_general(
            act, down_ref[0], (((1,), (1,)), ((), ())),
            preferred_element_type=jnp.float32)


def _ffn_call(be, nb, tok_pad, x_bf, egu, ed, interpret=False):
    spec = pltpu.PrefetchScalarGridSpec(
        num_scalar_prefetch=3,
        grid=(NBLK,),
        in_specs=[
            pl.BlockSpec((T, D), lambda b, be, nb, tok: (0, 0)),
            pl.BlockSpec((1, 2 * I, D), lambda b, be, nb, tok: (be[b], 0, 0),
                         pipeline_mode=pl.Buffered(buffer_count=1)),
            pl.BlockSpec((1, D, I), lambda b, be, nb, tok: (be[b], 0, 0),
                         pipeline_mode=pl.Buffered(buffer_count=1)),
        ],
        out_specs=pl.BlockSpec((BLK, D), lambda b, be, nb, tok: (b, 0)),
        scratch_shapes=[pltpu.VMEM((BLK, D), jnp.float32)],
    )
    return pl.pallas_call(
        _ffn_kernel,
        grid_spec=spec,
        out_shape=jax.ShapeDtypeStruct((NPAD, D), jnp.float32),
        interpret=interpret,
    )(be, nb, tok_pad, x_bf, egu, ed)


def _combine_kernel(pos_ref, twb_ref, y_ref, out_ref):
    tb = pl.program_id(0)

    def body(r2, _):
        for k in range(2):
            r = 2 * r2 + k
            g = tb * TB + r
            p0 = pos_ref[2 * g]
            p1 = pos_ref[2 * g + 1]
            w0 = jax.lax.bitcast_convert_type(twb_ref[2 * g], jnp.float32)
            w1 = jax.lax.bitcast_convert_type(twb_ref[2 * g + 1], jnp.float32)
            out_ref[pl.ds(r, 1), :] = (y_ref[pl.ds(p0, 1), :] * w0
                                       + y_ref[pl.ds(p1, 1), :] * w1)
        return 0

    jax.lax.fori_loop(0, TB // 2, body, 0)


def _combine_call(pos_flat, tw_flat_bits, y_pad, interpret=False):
    spec = pltpu.PrefetchScalarGridSpec(
        num_scalar_prefetch=2,
        grid=(T // TB,),
        in_specs=[pl.BlockSpec((NPAD, D), lambda tb, pos, tw: (0, 0))],
        out_specs=pl.BlockSpec((TB, D), lambda tb, pos, tw: (tb, 0)),
    )
    return pl.pallas_call(
        _combine_kernel,
        grid_spec=spec,
        out_shape=jax.ShapeDtypeStruct((T, D), jnp.float32),
        interpret=interpret,
    )(pos_flat, tw_flat_bits, y_pad)


def _moe(x, w_gate, egu, ed, interpret=False):
    b_, s_, d_ = x.shape
    x_flat = x.reshape(T, D)
    (ti, tw, cnt, pos, aux, be2, nb2) = _router_call(x_flat, w_gate,
                                                     interpret=interpret)
    be = be2.reshape(NBLK)
    nblk_act = nb2.reshape(1)
    pos_flat = pos.reshape(NP)
    tok_src = (jnp.arange(NP, dtype=jnp.int32) // K)
    tok_pad = jnp.zeros((NPAD,), jnp.int32).at[pos_flat].set(tok_src)

    y_pad = _ffn_call(be, nblk_act, tok_pad, x_flat, egu, ed,
                      interpret=interpret)

    tw_bits = jax.lax.bitcast_convert_type(tw.reshape(NP), jnp.int32)
    out = _combine_call(pos_flat, tw_bits, y_pad, interpret=interpret)
    return out.reshape(b_, s_, d_), aux[0, 0]


def kernel(x, W_gate, expert_gate_up, expert_down):
    return _moe(x, W_gate, expert_gate_up, expert_down)


# speedup vs baseline: 1.4373x; 1.0719x over previous
import functools

import jax
import jax.numpy as jnp
from jax.experimental import pallas as pl
from jax.experimental.pallas import tpu as pltpu

T = 2048
D = 1024
I = 2048
E = 8
K = 2
NP = T * K
BLK = 256
NBLK = NP // BLK + E - 1
NPAD = NBLK * BLK
IC = 512
NIC = I // IC
CH = 256
TB = 256


def _router_kernel(x_ref, wg_ref, ti_ref, tw_ref, cnt_ref, pos_ref, aux_ref,
                   be_ref, nb_ref, ohs_ref):
    x = x_ref[...]
    wg = wg_ref[...]
    logits = jax.lax.dot_general(x, wg, (((1,), (1,)), ((), ())),
                                 preferred_element_type=jnp.float32)
    m = jnp.max(logits, axis=1, keepdims=True)
    ex = jnp.exp(logits - m)
    probs = ex / jnp.sum(ex, axis=1, keepdims=True)

    eids = jax.lax.broadcasted_iota(jnp.int32, (T, E), 1)
    m1 = jnp.max(probs, axis=1, keepdims=True)
    i1 = jnp.min(jnp.where(probs >= m1, eids, E), axis=1, keepdims=True)
    oh1 = (eids == i1)
    probs2 = jnp.where(oh1, -jnp.inf, probs)
    m2 = jnp.max(probs2, axis=1, keepdims=True)
    i2 = jnp.min(jnp.where(probs2 >= m2, eids, E), axis=1, keepdims=True)
    oh2 = (eids == i2)
    denom = m1 + m2 + 1e-9
    w1 = m1 / denom
    w2 = m2 / denom

    oh1f = oh1.astype(jnp.float32)
    oh2f = oh2.astype(jnp.float32)
    ohs = oh1f + oh2f
    cnt = jnp.sum(ohs, axis=0, keepdims=True)
    p_mean = jnp.mean(probs, axis=0, keepdims=True)
    aux_ref[...] = E * jnp.sum((cnt / T) * p_mean, axis=1, keepdims=True)
    cnt_i = cnt.astype(jnp.int32)
    cnt_ref[...] = cnt_i

    pcnt = ((cnt + (BLK - 1)) // BLK).astype(jnp.float32) * BLK
    r8 = jax.lax.broadcasted_iota(jnp.int32, (E, E), 0)
    c8 = jax.lax.broadcasted_iota(jnp.int32, (E, E), 1)
    lt8 = (r8 < c8).astype(jnp.float32)
    pad_off = jax.lax.dot_general(pcnt, lt8, (((1,), (0,)), ((), ())),
                                  preferred_element_type=jnp.float32)

    ohs_ref[...] = ohs
    rr = jax.lax.broadcasted_iota(jnp.int32, (CH, CH), 0)
    cc = jax.lax.broadcasted_iota(jnp.int32, (CH, CH), 1)
    ltri = (cc < rr).astype(jnp.float32)

    def scan_body(c, carry):
        blk = ohs_ref[pl.ds(c * CH, CH), :]
        pre = jax.lax.dot_general(ltri, blk, (((1,), (0,)), ((), ())),
                                  preferred_element_type=jnp.float32)
        ohs_ref[pl.ds(c * CH, CH), :] = pre + carry
        return carry + jnp.sum(blk, axis=0, keepdims=True)

    jax.lax.fori_loop(0, T // CH, scan_body, jnp.zeros((1, E), jnp.float32))
    cex = ohs_ref[...]

    sel_off1 = jnp.sum(oh1f * pad_off, axis=1, keepdims=True)
    sel_off2 = jnp.sum(oh2f * pad_off, axis=1, keepdims=True)
    rank1 = jnp.sum(oh1f * cex, axis=1, keepdims=True)
    rank2 = jnp.sum(oh2f * cex, axis=1, keepdims=True)
    pos1 = (sel_off1 + rank1).astype(jnp.int32)
    pos2 = (sel_off2 + rank2).astype(jnp.int32)

    ti_ref[...] = jnp.concatenate([i1, i2], axis=1)
    tw_ref[...] = jnp.concatenate([w1, w2], axis=1)
    pos_ref[...] = jnp.concatenate([pos1, pos2], axis=1)

    le8 = (r8 <= c8).astype(jnp.float32)
    ends = jax.lax.dot_general(pcnt / BLK, le8, (((1,), (0,)), ((), ())),
                               preferred_element_type=jnp.float32)
    nbf = ends[:, E - 1:E]
    ends_b = jnp.broadcast_to(ends, (NBLK, E))
    bid_col = jax.lax.broadcasted_iota(jnp.int32, (NBLK, 1), 0).astype(
        jnp.float32)
    eb = jnp.sum((ends_b <= bid_col).astype(jnp.float32), axis=1,
                 keepdims=True)
    active = bid_col < nbf
    last_e = jnp.max(jnp.where(active, eb, -1.0), axis=0, keepdims=True)
    be_ref[...] = jnp.where(active, eb, last_e).astype(jnp.int32)
    nb_ref[...] = nbf.astype(jnp.int32)


def _router_call(x_flat, w_gate, interpret=False):
    return pl.pallas_call(
        _router_kernel,
        out_shape=[
            jax.ShapeDtypeStruct((T, K), jnp.int32),
            jax.ShapeDtypeStruct((T, K), jnp.float32),
            jax.ShapeDtypeStruct((1, E), jnp.int32),
            jax.ShapeDtypeStruct((T, K), jnp.int32),
            jax.ShapeDtypeStruct((1, 1), jnp.float32),
            jax.ShapeDtypeStruct((NBLK, 1), jnp.int32),
            jax.ShapeDtypeStruct((1, 1), jnp.int32),
        ],
        scratch_shapes=[pltpu.VMEM((T, E), jnp.float32)],
        interpret=interpret,
    )(x_flat, w_gate)


def _ffn_kernel(be_ref, nb_ref, tok_ref, x_ref, gu_ref, down_ref,
                y_ref, xg_ref):
    b = pl.program_id(0)

    @pl.when(b < nb_ref[0])
    def _active():
        def body(r2, _):
            for k in range(2):
                r = 2 * r2 + k
                t = tok_ref[b * BLK + r]
                xg_ref[pl.ds(r, 1), :] = x_ref[pl.ds(t, 1), :]
            return 0
        jax.lax.fori_loop(0, BLK // 2, body, 0)

        xg = xg_ref[...]
        h = jax.lax.dot_general(xg, gu_ref[0], (((1,), (1,)), ((), ())),
                                preferred_element_type=jnp.float32)
        g = h[:, :I]
        u = h[:, I:]
        act = (g / (1.0 + jnp.exp(-g))) * u
        y_ref[...] = jax.lax.dot_general(
            act, down_ref[0], (((1,), (1,)), ((), ())),
            preferred_element_type=jnp.float32)


def _ffn_call(be, nb, tok_pad, x_bf, egu, ed, interpret=False):
    spec = pltpu.PrefetchScalarGridSpec(
        num_scalar_prefetch=3,
        grid=(NBLK,),
        in_specs=[
            pl.BlockSpec((T, D), lambda b, be, nb, tok: (0, 0)),
            pl.BlockSpec((1, 2 * I, D), lambda b, be, nb, tok: (be[b], 0, 0),
                         pipeline_mode=pl.Buffered(buffer_count=1)),
            pl.BlockSpec((1, D, I), lambda b, be, nb, tok: (be[b], 0, 0),
                         pipeline_mode=pl.Buffered(buffer_count=1)),
        ],
        out_specs=pl.BlockSpec((BLK, D), lambda b, be, nb, tok: (b, 0)),
        scratch_shapes=[pltpu.VMEM((BLK, D), jnp.float32)],
    )
    return pl.pallas_call(
        _ffn_kernel,
        grid_spec=spec,
        out_shape=jax.ShapeDtypeStruct((NPAD, D), jnp.float32),
        interpret=interpret,
    )(be, nb, tok_pad, x_bf, egu, ed)


def _combine_kernel(pos_ref, twb_ref, y_ref, out_ref):
    tb = pl.program_id(0)

    def body(r2, _):
        for k in range(4):
            r = 4 * r2 + k
            g = tb * TB + r
            p0 = pos_ref[2 * g]
            p1 = pos_ref[2 * g + 1]
            w0 = jax.lax.bitcast_convert_type(twb_ref[2 * g], jnp.float32)
            w1 = jax.lax.bitcast_convert_type(twb_ref[2 * g + 1], jnp.float32)
            out_ref[pl.ds(r, 1), :] = (y_ref[pl.ds(p0, 1), :] * w0
                                       + y_ref[pl.ds(p1, 1), :] * w1)
        return 0

    jax.lax.fori_loop(0, TB // 4, body, 0)


def _combine_call(pos_flat, tw_flat_bits, y_pad, interpret=False):
    spec = pltpu.PrefetchScalarGridSpec(
        num_scalar_prefetch=2,
        grid=(T // TB,),
        in_specs=[pl.BlockSpec((NPAD, D), lambda tb, pos, tw: (0, 0))],
        out_specs=pl.BlockSpec((TB, D), lambda tb, pos, tw: (tb, 0)),
    )
    return pl.pallas_call(
        _combine_kernel,
        grid_spec=spec,
        out_shape=jax.ShapeDtypeStruct((T, D), jnp.float32),
        interpret=interpret,
    )(pos_flat, tw_flat_bits, y_pad)


def _moe(x, w_gate, egu, ed, interpret=False):
    b_, s_, d_ = x.shape
    x_flat = x.reshape(T, D)
    (ti, tw, cnt, pos, aux, be2, nb2) = _router_call(x_flat, w_gate,
                                                     interpret=interpret)
    be = be2.reshape(NBLK)
    nblk_act = nb2.reshape(1)
    pos_flat = pos.reshape(NP)
    tok_src = (jnp.arange(NP, dtype=jnp.int32) // K)
    tok_pad = jnp.zeros((NPAD,), jnp.int32).at[pos_flat].set(tok_src)

    y_pad = _ffn_call(be, nblk_act, tok_pad, x_flat, egu, ed,
                      interpret=interpret)

    tw_bits = jax.lax.bitcast_convert_type(tw.reshape(NP), jnp.int32)
    out = _combine_call(pos_flat, tw_bits, y_pad, interpret=interpret)
    return out.reshape(b_, s_, d_), aux[0, 0]


def kernel(x, W_gate, expert_gate_up, expert_down):
    return _moe(x, W_gate, expert_gate_up, expert_down)


# speedup vs baseline: 1.4683x; 1.0216x over previous
import functools

import jax
import jax.numpy as jnp
from jax.experimental import pallas as pl
from jax.experimental.pallas import tpu as pltpu

T = 2048
D = 1024
I = 2048
E = 8
K = 2
NP = T * K
BLK = 256
NBLK = NP // BLK + E - 1
NPAD = NBLK * BLK
IC = 512
NIC = I // IC
CH = 256
TB = 256


def _router_kernel(x_ref, wg_ref, ti_ref, tw_ref, cnt_ref, pos_ref, aux_ref,
                   be_ref, nb_ref, ohs_ref):
    x = x_ref[...]
    wg = wg_ref[...]
    logits = jax.lax.dot_general(x, wg, (((1,), (1,)), ((), ())),
                                 preferred_element_type=jnp.float32)
    m = jnp.max(logits, axis=1, keepdims=True)
    ex = jnp.exp(logits - m)
    probs = ex / jnp.sum(ex, axis=1, keepdims=True)

    eids = jax.lax.broadcasted_iota(jnp.int32, (T, E), 1)
    m1 = jnp.max(probs, axis=1, keepdims=True)
    i1 = jnp.min(jnp.where(probs >= m1, eids, E), axis=1, keepdims=True)
    oh1 = (eids == i1)
    probs2 = jnp.where(oh1, -jnp.inf, probs)
    m2 = jnp.max(probs2, axis=1, keepdims=True)
    i2 = jnp.min(jnp.where(probs2 >= m2, eids, E), axis=1, keepdims=True)
    oh2 = (eids == i2)
    denom = m1 + m2 + 1e-9
    w1 = m1 / denom
    w2 = m2 / denom

    oh1f = oh1.astype(jnp.float32)
    oh2f = oh2.astype(jnp.float32)
    ohs = oh1f + oh2f
    cnt = jnp.sum(ohs, axis=0, keepdims=True)
    p_mean = jnp.mean(probs, axis=0, keepdims=True)
    aux_ref[...] = E * jnp.sum((cnt / T) * p_mean, axis=1, keepdims=True)
    cnt_i = cnt.astype(jnp.int32)
    cnt_ref[...] = cnt_i

    pcnt = ((cnt + (BLK - 1)) // BLK).astype(jnp.float32) * BLK
    r8 = jax.lax.broadcasted_iota(jnp.int32, (E, E), 0)
    c8 = jax.lax.broadcasted_iota(jnp.int32, (E, E), 1)
    lt8 = (r8 < c8).astype(jnp.float32)
    pad_off = jax.lax.dot_general(pcnt, lt8, (((1,), (0,)), ((), ())),
                                  preferred_element_type=jnp.float32)

    ohs_ref[...] = ohs
    rr = jax.lax.broadcasted_iota(jnp.int32, (CH, CH), 0)
    cc = jax.lax.broadcasted_iota(jnp.int32, (CH, CH), 1)
    ltri = (cc < rr).astype(jnp.float32)

    def scan_body(c, carry):
        blk = ohs_ref[pl.ds(c * CH, CH), :]
        pre = jax.lax.dot_general(ltri, blk, (((1,), (0,)), ((), ())),
                                  preferred_element_type=jnp.float32)
        ohs_ref[pl.ds(c * CH, CH), :] = pre + carry
        return carry + jnp.sum(blk, axis=0, keepdims=True)

    jax.lax.fori_loop(0, T // CH, scan_body, jnp.zeros((1, E), jnp.float32))
    cex = ohs_ref[...]

    sel_off1 = jnp.sum(oh1f * pad_off, axis=1, keepdims=True)
    sel_off2 = jnp.sum(oh2f * pad_off, axis=1, keepdims=True)
    rank1 = jnp.sum(oh1f * cex, axis=1, keepdims=True)
    rank2 = jnp.sum(oh2f * cex, axis=1, keepdims=True)
    pos1 = (sel_off1 + rank1).astype(jnp.int32)
    pos2 = (sel_off2 + rank2).astype(jnp.int32)

    ti_ref[...] = jnp.concatenate([i1, i2], axis=1)
    tw_ref[...] = jnp.concatenate([w1, w2], axis=1)
    pos_ref[...] = jnp.concatenate([pos1, pos2], axis=1)

    le8 = (r8 <= c8).astype(jnp.float32)
    ends = jax.lax.dot_general(pcnt / BLK, le8, (((1,), (0,)), ((), ())),
                               preferred_element_type=jnp.float32)
    nbf = ends[:, E - 1:E]
    ends_b = jnp.broadcast_to(ends, (NBLK, E))
    bid_col = jax.lax.broadcasted_iota(jnp.int32, (NBLK, 1), 0).astype(
        jnp.float32)
    eb = jnp.sum((ends_b <= bid_col).astype(jnp.float32), axis=1,
                 keepdims=True)
    active = bid_col < nbf
    last_e = jnp.max(jnp.where(active, eb, -1.0), axis=0, keepdims=True)
    be_ref[...] = jnp.where(active, eb, last_e).astype(jnp.int32)
    nb_ref[...] = nbf.astype(jnp.int32)


def _router_call(x_flat, w_gate, interpret=False):
    return pl.pallas_call(
        _router_kernel,
        out_shape=[
            jax.ShapeDtypeStruct((T, K), jnp.int32),
            jax.ShapeDtypeStruct((T, K), jnp.float32),
            jax.ShapeDtypeStruct((1, E), jnp.int32),
            jax.ShapeDtypeStruct((T, K), jnp.int32),
            jax.ShapeDtypeStruct((1, 1), jnp.float32),
            jax.ShapeDtypeStruct((NBLK, 1), jnp.int32),
            jax.ShapeDtypeStruct((1, 1), jnp.int32),
        ],
        scratch_shapes=[pltpu.VMEM((T, E), jnp.float32)],
        interpret=interpret,
    )(x_flat, w_gate)


def _ffn_kernel(be_ref, nb_ref, tok_ref, x_ref, gu_ref, down_ref,
                y_ref, xg_ref):
    b = pl.program_id(0)

    @pl.when(b < nb_ref[0])
    def _active():
        def body(r2, _):
            for k in range(4):
                r = 4 * r2 + k
                t = tok_ref[b * BLK + r]
                xg_ref[pl.ds(r, 1), :] = x_ref[pl.ds(t, 1), :]
            return 0
        jax.lax.fori_loop(0, BLK // 4, body, 0)

        xg = xg_ref[...]
        h = jax.lax.dot_general(xg, gu_ref[0], (((1,), (1,)), ((), ())),
                                preferred_element_type=jnp.float32)
        g = h[:, :I]
        u = h[:, I:]
        act = (g / (1.0 + jnp.exp(-g))) * u
        y_ref[...] = jax.lax.dot_general(
            act, down_ref[0], (((1,), (1,)), ((), ())),
            preferred_element_type=jnp.float32)


def _ffn_call(be, nb, tok_pad, x_bf, egu, ed, interpret=False):
    spec = pltpu.PrefetchScalarGridSpec(
        num_scalar_prefetch=3,
        grid=(NBLK,),
        in_specs=[
            pl.BlockSpec((T, D), lambda b, be, nb, tok: (0, 0)),
            pl.BlockSpec((1, 2 * I, D), lambda b, be, nb, tok: (be[b], 0, 0),
                         pipeline_mode=pl.Buffered(buffer_count=1)),
            pl.BlockSpec((1, D, I), lambda b, be, nb, tok: (be[b], 0, 0),
                         pipeline_mode=pl.Buffered(buffer_count=1)),
        ],
        out_specs=pl.BlockSpec((BLK, D), lambda b, be, nb, tok: (b, 0)),
        scratch_shapes=[pltpu.VMEM((BLK, D), jnp.float32)],
    )
    return pl.pallas_call(
        _ffn_kernel,
        grid_spec=spec,
        out_shape=jax.ShapeDtypeStruct((NPAD, D), jnp.float32),
        interpret=interpret,
    )(be, nb, tok_pad, x_bf, egu, ed)


def _combine_kernel(pos_ref, twb_ref, y_ref, out_ref):
    tb = pl.program_id(0)

    def body(r2, _):
        for k in range(8):
            r = 8 * r2 + k
            g = tb * TB + r
            p0 = pos_ref[2 * g]
            p1 = pos_ref[2 * g + 1]
            w0 = jax.lax.bitcast_convert_type(twb_ref[2 * g], jnp.float32)
            w1 = jax.lax.bitcast_convert_type(twb_ref[2 * g + 1], jnp.float32)
            out_ref[pl.ds(r, 1), :] = (y_ref[pl.ds(p0, 1), :] * w0
                                       + y_ref[pl.ds(p1, 1), :] * w1)
        return 0

    jax.lax.fori_loop(0, TB // 8, body, 0)


def _combine_call(pos_flat, tw_flat_bits, y_pad, interpret=False):
    spec = pltpu.PrefetchScalarGridSpec(
        num_scalar_prefetch=2,
        grid=(T // TB,),
        in_specs=[pl.BlockSpec((NPAD, D), lambda tb, pos, tw: (0, 0))],
        out_specs=pl.BlockSpec((TB, D), lambda tb, pos, tw: (tb, 0)),
    )
    return pl.pallas_call(
        _combine_kernel,
        grid_spec=spec,
        out_shape=jax.ShapeDtypeStruct((T, D), jnp.float32),
        interpret=interpret,
    )(pos_flat, tw_flat_bits, y_pad)


def _moe(x, w_gate, egu, ed, interpret=False):
    b_, s_, d_ = x.shape
    x_flat = x.reshape(T, D)
    (ti, tw, cnt, pos, aux, be2, nb2) = _router_call(x_flat, w_gate,
                                                     interpret=interpret)
    be = be2.reshape(NBLK)
    nblk_act = nb2.reshape(1)
    pos_flat = pos.reshape(NP)
    tok_src = (jnp.arange(NP, dtype=jnp.int32) // K)
    tok_pad = jnp.zeros((NPAD,), jnp.int32).at[pos_flat].set(tok_src)

    y_pad = _ffn_call(be, nblk_act, tok_pad, x_flat, egu, ed,
                      interpret=interpret)

    tw_bits = jax.lax.bitcast_convert_type(tw.reshape(NP), jnp.int32)
    out = _combine_call(pos_flat, tw_bits, y_pad, interpret=interpret)
    return out.reshape(b_, s_, d_), aux[0, 0]


def kernel(x, W_gate, expert_gate_up, expert_down):
    return _moe(x, W_gate, expert_gate_up, expert_down)
